# baseline (speedup 1.0000x reference)
import functools

import jax
import jax.numpy as jnp
from jax import lax
from jax.experimental import pallas as pl
from jax.experimental.pallas import tpu as pltpu
from jax.experimental.pallas import tpu_sc as plsc

_ROWS = 16384
_D = 128
_KEEP = 64

_VROWS = _ROWS // 2
_VCOLS = _D * 2

_info = plsc.get_sparse_core_info()
_NC = _info.num_cores
_NS = _info.num_subcores
_NW = _NC * _NS
_VROWS_PER_W = _VROWS // _NW

_NCHUNK = 4
_CHUNK = _VROWS_PER_W // _NCHUNK

_mesh = plsc.VectorSubcoreMesh(core_axis_name="c", subcore_axis_name="s")


@functools.partial(
    pl.kernel,
    mesh=_mesh,
    out_type=jax.ShapeDtypeStruct((_VROWS, _KEEP * 2), jnp.float32),
    scratch_types=[
        pltpu.VMEM((_VROWS_PER_W, _KEEP * 2), jnp.float32),
        [pltpu.SemaphoreType.DMA] * _NCHUNK,
        pltpu.SemaphoreType.DMA,
    ],
    compiler_params=pltpu.CompilerParams(
        use_tc_tiling_on_sc=False,
        disable_bounds_checks=True,
        disable_semaphore_checks=True,
    ),
)
def _mask_copy(x_hbm, out_hbm, buf, in_sems, out_sem):
    wid = lax.axis_index("s") * _NC + lax.axis_index("c")
    base = wid * _VROWS_PER_W
    ins = []
    for k in range(_NCHUNK):
        lo = base + k * _CHUNK
        dst = buf.at[pl.ds(k * _CHUNK, _CHUNK)]
        ins.append(
            (
                pltpu.async_copy(
                    x_hbm.at[pl.ds(lo, _CHUNK), pl.ds(0, _KEEP)],
                    dst.at[:, pl.ds(0, _KEEP)],
                    in_sems[k],
                ),
                pltpu.async_copy(
                    x_hbm.at[pl.ds(lo, _CHUNK), pl.ds(_D, _KEEP)],
                    dst.at[:, pl.ds(_KEEP, _KEEP)],
                    in_sems[k],
                ),
            )
        )
    outs = []
    for k in range(_NCHUNK):
        for cp in ins[k]:
            cp.wait()
        outs.append(
            pltpu.async_copy(
                buf.at[pl.ds(k * _CHUNK, _CHUNK)],
                out_hbm.at[pl.ds(base + k * _CHUNK, _CHUNK)],
                out_sem,
            )
        )
    for cp in outs:
        cp.wait()


def kernel(inputs):
    out = _mask_copy(inputs.reshape(_VROWS, _VCOLS))
    return out.reshape(_ROWS, _KEEP)


# speedup vs baseline: 1.1702x; 1.1702x over previous
import functools

import jax
import jax.numpy as jnp
from jax import lax
from jax.experimental import pallas as pl
from jax.experimental.pallas import tpu as pltpu
from jax.experimental.pallas import tpu_sc as plsc

_ROWS = 16384
_D = 128
_KEEP = 64

_info = plsc.get_sparse_core_info()
_NC = _info.num_cores
_NS = _info.num_subcores
_NW = _NC * _NS
_ROWS_PER_W = _ROWS // _NW

_NCHUNK = 4
_CHUNK = _ROWS_PER_W // _NCHUNK
_LANES = 16
_KVECS = _KEEP // _LANES

_mesh = plsc.VectorSubcoreMesh(core_axis_name="c", subcore_axis_name="s")


@functools.partial(
    pl.kernel,
    mesh=_mesh,
    out_type=jax.ShapeDtypeStruct((_ROWS, _KEEP), jnp.float32),
    scratch_types=[
        [pltpu.VMEM((_CHUNK, _D), jnp.float32)] * _NCHUNK,
        [pltpu.VMEM((_CHUNK, _KEEP), jnp.float32)] * 2,
        [pltpu.SemaphoreType.DMA] * _NCHUNK,
        [pltpu.SemaphoreType.DMA] * 2,
    ],
    compiler_params=pltpu.CompilerParams(
        disable_bounds_checks=True,
        disable_semaphore_checks=True,
    ),
)
def _mask_copy(x_hbm, out_hbm, in_bufs, out_bufs, in_sems, out_sems):
    wid = lax.axis_index("s") * _NC + lax.axis_index("c")
    base = wid * _ROWS_PER_W
    ins = []
    for k in range(_NCHUNK):
        ins.append(
            pltpu.async_copy(
                x_hbm.at[pl.ds(base + k * _CHUNK, _CHUNK)],
                in_bufs[k],
                in_sems[k],
            )
        )
    outs = {}
    for k in range(_NCHUNK):
        s = k % 2
        ins[k].wait()
        if k >= 2:
            outs[k - 2].wait()
        ib = in_bufs[k]
        ob = out_bufs[s]

        def body(r, _, ib=ib, ob=ob):
            for c in range(_KVECS):
                ob[r, pl.ds(c * _LANES, _LANES)] = ib[
                    r, pl.ds(c * _LANES, _LANES)
                ]
            return 0

        lax.fori_loop(0, _CHUNK, body, 0)
        outs[k] = pltpu.async_copy(
            ob,
            out_hbm.at[pl.ds(base + k * _CHUNK, _CHUNK)],
            out_sems[s],
        )
    outs[_NCHUNK - 2].wait()
    outs[_NCHUNK - 1].wait()


def kernel(inputs):
    return _mask_copy(inputs)
